# baseline (speedup 1.0000x reference)
import functools

import jax
import jax.numpy as jnp
from jax.experimental import pallas as pl
from jax.experimental.pallas import tpu as pltpu

_B, _N = 64, 1000000
_BC = 4096
_GRID = (_N + _BC - 1) // _BC


def _threefry_bits(flat_u32):
    ks0 = jnp.uint32(0)
    ks1 = jnp.uint32(42)
    ks2 = jnp.uint32(0 ^ 42 ^ 0x1BD11BDA)
    ks = (ks0, ks1, ks2)
    rot = ((13, 15, 26, 6), (17, 29, 16, 24))
    x0 = jnp.zeros_like(flat_u32) + ks0
    x1 = flat_u32 + ks1

    def rotl(x, d):
        return (x << jnp.uint32(d)) | (x >> jnp.uint32(32 - d))

    for i in range(5):
        for d in rot[i % 2]:
            x0 = x0 + x1
            x1 = rotl(x1, d)
            x1 = x0 ^ x1
        x0 = x0 + ks[(i + 1) % 3]
        x1 = x1 + ks[(i + 2) % 3] + jnp.uint32(i + 1)
    return x0 ^ x1


def _kernel(x_ref, o_ref, best_ref, bidx_ref):
    i = pl.program_id(0)

    @pl.when(i == 0)
    def _init():
        best_ref[...] = jnp.full_like(best_ref, -jnp.inf)
        bidx_ref[...] = jnp.zeros_like(bidx_ref)

    col = jax.lax.broadcasted_iota(jnp.int32, (_B, _BC), 1) + i * _BC
    row = jax.lax.broadcasted_iota(jnp.int32, (_B, _BC), 0)
    flat = (row * _N + col).astype(jnp.uint32)
    bits = _threefry_bits(flat)

    fb = (bits >> jnp.uint32(9)) | jnp.uint32(0x3F800000)
    floats = pltpu.bitcast(fb, jnp.float32) - jnp.float32(1.0)
    u = jnp.maximum(jnp.float32(1e-20),
                    floats * jnp.float32(1.0 - 1e-20) + jnp.float32(1e-20))
    g = -jnp.log(-jnp.log(u))
    v = x_ref[...] + g
    v = jnp.where(col < _N, v, -jnp.inf)

    m = jnp.max(v, axis=1, keepdims=True)
    idx = jnp.min(jnp.where(v == m, col, jnp.int32(2**31 - 1)),
                  axis=1, keepdims=True)

    better = m > best_ref[...]
    old_best = best_ref[...]
    old_idx = bidx_ref[...]
    best_ref[...] = jnp.where(better, m, old_best)
    bidx_ref[...] = jnp.where(better, idx, old_idx)

    @pl.when(i == _GRID - 1)
    def _done():
        o_ref[...] = bidx_ref[...].astype(jnp.float32)


@jax.jit
def kernel(inputs):
    out = pl.pallas_call(
        _kernel,
        grid=(_GRID,),
        in_specs=[pl.BlockSpec((_B, _BC), lambda i: (0, i))],
        out_specs=pl.BlockSpec((_B, 1), lambda i: (0, 0)),
        out_shape=jax.ShapeDtypeStruct((_B, 1), jnp.float32),
        scratch_shapes=[
            pltpu.VMEM((_B, 1), jnp.float32),
            pltpu.VMEM((_B, 1), jnp.int32),
        ],
        compiler_params=pltpu.CompilerParams(
            dimension_semantics=("arbitrary",),
        ),
    )(inputs)
    return out
